# baseline (speedup 1.0000x reference)
import jax
import jax.numpy as jnp
from jax import lax
from jax.experimental import pallas as pl
from jax.experimental.pallas import tpu as pltpu
from jax.experimental.pallas import tpu_sc as plsc

H, W, C = 512, 512, 63
NCORES, NSUB = 2, 16
NW = NCORES * NSUB
RPW = H // NW
LANES = 16
HALF = 256
UNITS = RPW * (W // HALF)
CSTORE = 64
CHUNKS = HALF // LANES


def _body(img_hbm, out_hbm, rowv0, rowv1, buf0, buf1, sem0, sem1):
    wid = lax.axis_index("s") * NCORES + lax.axis_index("c")
    r_base = wid * RPW
    iota = lax.iota(jnp.int32, LANES)
    rowvs = (rowv0, rowv1)
    bufs = (buf0, buf1)
    sems = (sem0, sem1)

    def img_at(u):
        return img_hbm.at[0, r_base + u // 2, pl.ds((u % 2) * HALF, HALF)]

    def out_at(u):
        return out_hbm.at[r_base + u // 2, pl.ds((u % 2) * HALF, HALF)]

    def compute_unit(rowv, buf):
        @pl.loop(0, CHUNKS)
        def _(jb):
            base = jb * LANES
            for lane in range(LANES):
                pix = base + lane
                splat = plsc.load_gather(
                    rowv, [jnp.full((LANES,), pix, jnp.int32)])
                for k0 in (0, LANES, 2 * LANES, C - LANES):
                    m = splat == (iota + (k0 + 1))
                    buf[pix, pl.ds(k0, LANES)] = m.astype(jnp.int32)

    for b in range(2):
        pltpu.sync_copy(img_at(b), rowvs[b])
        compute_unit(rowvs[b], bufs[b])
        pltpu.async_copy(bufs[b], out_at(b), sems[b])

    @pl.loop(2, UNITS, step=2)
    def _(u):
        for b in range(2):
            pltpu.make_async_copy(
                bufs[b], out_at(u + b - 2), sems[b]).wait()
            pltpu.sync_copy(img_at(u + b), rowvs[b])
            compute_unit(rowvs[b], bufs[b])
            pltpu.async_copy(bufs[b], out_at(u + b), sems[b])

    for b in range(2):
        pltpu.make_async_copy(
            bufs[b], out_at(UNITS - 2 + b), sems[b]).wait()


@jax.jit
def _onehot(img):
    run = pl.kernel(
        _body,
        out_type=jax.ShapeDtypeStruct((H, W, C), jnp.int32),
        mesh=plsc.VectorSubcoreMesh(core_axis_name="c", subcore_axis_name="s"),
        compiler_params=pltpu.CompilerParams(
            needs_layout_passes=False, use_tc_tiling_on_sc=True),
        scratch_types=[
            pltpu.VMEM((HALF,), jnp.int32),
            pltpu.VMEM((HALF,), jnp.int32),
            pltpu.VMEM((HALF, C), jnp.int32),
            pltpu.VMEM((HALF, C), jnp.int32),
            pltpu.SemaphoreType.DMA,
            pltpu.SemaphoreType.DMA,
        ],
    )
    return run(img)


def kernel(img):
    return _onehot(img)


# speedup vs baseline: 1.1940x; 1.1940x over previous
import jax
import jax.numpy as jnp
from jax.experimental import pallas as pl
from jax.experimental.pallas import tpu as pltpu

H, W, C = 512, 512, 63
BR = 8


def _tc_body(img_ref, out_ref):
    v = img_ref[0]
    cls = jax.lax.broadcasted_iota(jnp.int32, (1, 1, C), 2) + 1
    out_ref[...] = (v[:, :, None] == cls).astype(jnp.int32)


@jax.jit
def _onehot(img):
    return pl.pallas_call(
        _tc_body,
        out_shape=jax.ShapeDtypeStruct((H, W, C), jnp.int32),
        grid=(H // BR,),
        in_specs=[pl.BlockSpec((1, BR, W), lambda i: (0, i, 0))],
        out_specs=pl.BlockSpec((BR, W, C), lambda i: (i, 0, 0)),
    )(img)


def kernel(img):
    return _onehot(img)


# speedup vs baseline: 1.7863x; 1.4960x over previous
import jax
import jax.numpy as jnp
from jax.experimental import pallas as pl

H, W, C = 512, 512, 63
BR = 8


def _tc_body(img_ref, out_ref):
    v = img_ref[0][:, None, :]
    cls = jax.lax.broadcasted_iota(jnp.int32, (1, C, 1), 1) + 1
    out_ref[...] = (v == cls).astype(jnp.int32)


@jax.jit
def _onehot(img):
    enc = pl.pallas_call(
        _tc_body,
        out_shape=jax.ShapeDtypeStruct((H, C, W), jnp.int32),
        grid=(H // BR,),
        in_specs=[pl.BlockSpec((1, BR, W), lambda i: (0, i, 0))],
        out_specs=pl.BlockSpec((BR, C, W), lambda i: (i, 0, 0)),
    )(img)
    return enc.transpose(0, 2, 1)


def kernel(img):
    return _onehot(img)


# speedup vs baseline: 4.4693x; 2.5020x over previous
import jax
import jax.numpy as jnp
from jax.experimental import pallas as pl

H, W, C = 512, 512, 63
BR = 8


def _tc_body(img_ref, out_ref):
    v = img_ref[0][None, :, :]
    cls = jax.lax.broadcasted_iota(jnp.int32, (C, 1, 1), 0) + 1
    out_ref[...] = (v == cls).astype(jnp.int32)


@jax.jit
def _onehot(img):
    enc = pl.pallas_call(
        _tc_body,
        out_shape=jax.ShapeDtypeStruct((C, H, W), jnp.int32),
        grid=(H // BR,),
        in_specs=[pl.BlockSpec((1, BR, W), lambda i: (0, i, 0))],
        out_specs=pl.BlockSpec((C, BR, W), lambda i: (0, i, 0)),
    )(img)
    return enc.transpose(1, 2, 0)


def kernel(img):
    return _onehot(img)


# speedup vs baseline: 5.6822x; 1.2714x over previous
import jax
import jax.numpy as jnp
from jax.experimental import pallas as pl

H, W, C = 512, 512, 63


def _tc_body(img_ref, out_ref):
    c = pl.program_id(0)
    out_ref[...] = (img_ref[...] == c + 1).astype(jnp.int32)


@jax.jit
def _onehot(img):
    enc = pl.pallas_call(
        _tc_body,
        out_shape=jax.ShapeDtypeStruct((C, H, W), jnp.int32),
        grid=(C,),
        in_specs=[pl.BlockSpec((1, H, W), lambda c: (0, 0, 0))],
        out_specs=pl.BlockSpec((1, H, W), lambda c: (c, 0, 0)),
    )(img)
    return enc.transpose(1, 2, 0)


def kernel(img):
    return _onehot(img)


# speedup vs baseline: 8.6453x; 1.5215x over previous
import jax
import jax.numpy as jnp
from jax.experimental import pallas as pl

H, W, C = 512, 512, 63
CB = 7


def _tc_body(img_ref, out_ref):
    c0 = pl.program_id(0) * CB
    cls = jax.lax.broadcasted_iota(jnp.int32, (CB, 1, 1), 0) + (c0 + 1)
    out_ref[...] = (img_ref[...] == cls).astype(jnp.int32)


@jax.jit
def _onehot(img):
    enc = pl.pallas_call(
        _tc_body,
        out_shape=jax.ShapeDtypeStruct((C, H, W), jnp.int32),
        grid=(C // CB,),
        in_specs=[pl.BlockSpec((1, H, W), lambda c: (0, 0, 0))],
        out_specs=pl.BlockSpec((CB, H, W), lambda c: (c, 0, 0)),
    )(img)
    return enc.transpose(1, 2, 0)


def kernel(img):
    return _onehot(img)
